# baseline (speedup 1.0000x reference)
import functools

import jax
import jax.numpy as jnp
from jax import lax
from jax.experimental import pallas as pl
from jax.experimental.pallas import tpu as pltpu
from jax.experimental.pallas import tpu_sc as plsc


def kernel(input):
    shape = input.shape
    n = input.size
    x1d = input.reshape(n)

    info = plsc.get_sparse_core_info()
    nc, ns = info.num_cores, info.num_subcores
    nw = nc * ns
    chunk = n // nw

    mesh = plsc.VectorSubcoreMesh(core_axis_name="c", subcore_axis_name="s")

    @functools.partial(
        pl.kernel,
        mesh=mesh,
        out_type=jax.ShapeDtypeStruct((n,), input.dtype),
        scratch_types=[pltpu.VMEM((chunk,), input.dtype)],
    )
    def sc_copy(x_hbm, o_hbm, buf):
        wid = lax.axis_index("s") * nc + lax.axis_index("c")
        base = wid * chunk
        pltpu.sync_copy(x_hbm.at[pl.ds(base, chunk)], buf)
        pltpu.sync_copy(buf, o_hbm.at[pl.ds(base, chunk)])

    return sc_copy(x1d).reshape(shape)
